# baseline (speedup 1.0000x reference)
import functools

import jax
import jax.numpy as jnp
from jax import lax
from jax.experimental import pallas as pl
from jax.experimental.pallas import tpu as pltpu
from jax.experimental.pallas import tpu_sc as plsc

_NC = 2
_NS = 16
_NW = _NC * _NS
_C = 2
_NBUF = 6

_LN2 = 0.6931471805599453
_SQRT2 = 1.4142135623730951


def _vlog(x):
    bits = plsc.bitcast(x, jnp.int32)
    e = ((bits >> 23) & 0xFF) - 127
    mant = plsc.bitcast((bits & 0x007FFFFF) | (127 << 23), jnp.float32)
    big = mant > jnp.float32(_SQRT2)
    mant = jnp.where(big, mant * jnp.float32(0.5), mant)
    e = (e + jnp.where(big, 1, 0)).astype(jnp.float32)
    t = mant - jnp.float32(1.0)
    w = t / (t + jnp.float32(2.0))
    w2 = w * w
    p = jnp.float32(1.0 / 7.0) + w2 * jnp.float32(1.0 / 9.0)
    p = jnp.float32(1.0 / 5.0) + w2 * p
    p = jnp.float32(1.0 / 3.0) + w2 * p
    p = jnp.float32(2.0) * w * (jnp.float32(1.0) + w2 * p)
    return e * jnp.float32(_LN2) + p


def _make_sc_gather(n_tok, vocab, width):
    n_per_w = n_tok // _NW
    n_chunks = n_per_w // _C
    nvec = width // 16
    mesh = plsc.VectorSubcoreMesh(core_axis_name="c", subcore_axis_name="s")

    @functools.partial(
        pl.kernel,
        out_type=[
            jax.ShapeDtypeStruct((n_tok, width), jnp.float32),
            jax.ShapeDtypeStruct((_NW, 16), jnp.float32),
            jax.ShapeDtypeStruct((_NW, 16), jnp.float32),
        ],
        mesh=mesh,
        scratch_types=[
            pltpu.VMEM((n_chunks, _C), jnp.int32),
            pltpu.VMEM((n_per_w,), jnp.int32),
            pltpu.VMEM((16,), jnp.float32),
            pltpu.VMEM((16,), jnp.float32),
            [pltpu.VMEM((_C, width), jnp.float32) for _ in range(_NBUF)],
            [pltpu.SemaphoreType.DMA for _ in range(_NBUF)],
            [pltpu.SemaphoreType.DMA for _ in range(_NBUF)],
        ],
        compiler_params=pltpu.CompilerParams(needs_layout_passes=False),
    )
    def sc_gather(table_hbm, idx_hbm, tgt_hbm,
                  out_hbm, lpart_hbm, tpart_hbm,
                  idx_v, tgt_v, pl_v, pv_v, bufs, gsems, ssems):
        cid = lax.axis_index("c")
        sid = lax.axis_index("s")
        wid = sid * _NC + cid
        base = wid * n_per_w

        pltpu.sync_copy(idx_hbm.at[wid], idx_v)
        pltpu.sync_copy(tgt_hbm.at[wid], tgt_v)
        pl_v[...] = jnp.zeros((16,), jnp.float32)
        pv_v[...] = jnp.zeros((16,), jnp.float32)

        lane = lax.iota(jnp.int32, 16)
        msk_c = lane < _C
        neg_big = jnp.full((16,), jnp.finfo(jnp.float32).min, jnp.float32)
        zeros = jnp.zeros((16,), jnp.float32)

        def start_gather(k, b):
            pltpu.async_copy(table_hbm.at[idx_v.at[k]], bufs[b], gsems[b])

        def wait_gather(b):
            pltpu.make_async_copy(
                out_hbm.at[pl.ds(base, _C)], bufs[b], gsems[b]).wait()

        def start_scatter(k, b):
            pltpu.async_copy(
                bufs[b], out_hbm.at[pl.ds(base + k * _C, _C)], ssems[b])

        def wait_scatter(b):
            pltpu.make_async_copy(
                bufs[b], out_hbm.at[pl.ds(base, _C)], ssems[b]).wait()

        def compute(k, b):
            buf = bufs[b]
            tok = jnp.minimum(k * _C + lane, n_per_w - 1)
            tgts = plsc.load_gather(tgt_v, [tok])
            rowl = jnp.minimum(lane, _C - 1)
            tv = plsc.load_gather(buf, [rowl, tgts])
            pv_v[...] += jnp.where(msk_c, tv, jnp.float32(0.0))

            def p1(j, ms):
                return tuple(
                    jnp.maximum(ms[r], buf[r, pl.ds(j * 16, 16)])
                    for r in range(_C))

            m = lax.fori_loop(0, nvec, p1, (neg_big,) * _C, unroll=8)

            def p2(j, ss):
                return tuple(
                    ss[r] + jnp.exp(buf[r, pl.ds(j * 16, 16)] - m[r])
                    for r in range(_C))

            s = lax.fori_loop(0, nvec, p2, (zeros,) * _C, unroll=8)

            mvec = zeros
            svec = jnp.full((16,), jnp.float32(1.0), jnp.float32)
            for r in range(_C):
                mr = jnp.max(m[r])
                sr = jnp.sum(s[r] * jnp.exp(m[r] - mr))
                mvec = jnp.where(lane == r, mr, mvec)
                svec = jnp.where(lane == r, sr, svec)
            lse = mvec + _vlog(svec)
            pl_v[...] += jnp.where(msk_c, lse, jnp.float32(0.0))

        for b in range(_NBUF):
            start_gather(b, b)

        def step(k, b, prev_wait, prev_gather):
            wait_gather(b)
            start_scatter(k, b)
            bp = (b + _NBUF - 1) % _NBUF
            if prev_wait:
                wait_scatter(bp)
            if prev_gather:
                start_gather(k - 1 + _NBUF, bp)
            compute(k, b)

        for k in range(_NBUF):
            step(k, k, prev_wait=k > 0, prev_gather=k > 0)

        def loop_body(go, carry):
            for b in range(_NBUF):
                k = go * _NBUF + b
                step(k, b, prev_wait=True, prev_gather=True)
            return carry

        n_tail = _NBUF + n_chunks % _NBUF
        lax.fori_loop(1, (n_chunks - n_tail) // _NBUF, loop_body, 0,
                      unroll=False)

        for k in range(n_chunks - n_tail, n_chunks):
            b = k % _NBUF
            step(k, b, prev_wait=True, prev_gather=(k - 1 + _NBUF) < n_chunks)
        wait_scatter((n_chunks - 1) % _NBUF)

        pltpu.sync_copy(pl_v, lpart_hbm.at[wid])
        pltpu.sync_copy(pv_v, tpart_hbm.at[wid])

    return sc_gather


def kernel(idx, targets, table):
    vocab, width = table.shape
    n_tok = idx.shape[0] * idx.shape[1]
    n_per_w = n_tok // _NW
    idx3 = idx.reshape(_NW, n_per_w // _C, _C).astype(jnp.int32)
    tgt2 = targets.reshape(_NW, n_per_w).astype(jnp.int32)

    logits2, lse_parts, tval_parts = _make_sc_gather(n_tok, vocab, width)(
        table, idx3, tgt2)
    loss = (jnp.sum(lse_parts) - jnp.sum(tval_parts)) / jnp.float32(n_tok)
    return (logits2, loss)


# speedup vs baseline: 1.0331x; 1.0331x over previous
import functools

import jax
import jax.numpy as jnp
from jax import lax
from jax.experimental import pallas as pl
from jax.experimental.pallas import tpu as pltpu
from jax.experimental.pallas import tpu_sc as plsc

_NC = 2
_NS = 16
_NW = _NC * _NS
_C = 4
_NBUF = 3

_LN2 = 0.6931471805599453
_SQRT2 = 1.4142135623730951


def _vlog(x):
    bits = plsc.bitcast(x, jnp.int32)
    e = ((bits >> 23) & 0xFF) - 127
    mant = plsc.bitcast((bits & 0x007FFFFF) | (127 << 23), jnp.float32)
    big = mant > jnp.float32(_SQRT2)
    mant = jnp.where(big, mant * jnp.float32(0.5), mant)
    e = (e + jnp.where(big, 1, 0)).astype(jnp.float32)
    t = mant - jnp.float32(1.0)
    w = t / (t + jnp.float32(2.0))
    w2 = w * w
    p = jnp.float32(1.0 / 7.0) + w2 * jnp.float32(1.0 / 9.0)
    p = jnp.float32(1.0 / 5.0) + w2 * p
    p = jnp.float32(1.0 / 3.0) + w2 * p
    p = jnp.float32(2.0) * w * (jnp.float32(1.0) + w2 * p)
    return e * jnp.float32(_LN2) + p


def _make_sc_gather(n_tok, vocab, width):
    n_per_w = n_tok // _NW
    n_chunks = n_per_w // _C
    nvec = width // 16
    mesh = plsc.VectorSubcoreMesh(core_axis_name="c", subcore_axis_name="s")

    @functools.partial(
        pl.kernel,
        out_type=[
            jax.ShapeDtypeStruct((n_tok, width), jnp.float32),
            jax.ShapeDtypeStruct((_NW, 16), jnp.float32),
            jax.ShapeDtypeStruct((_NW, 16), jnp.float32),
        ],
        mesh=mesh,
        scratch_types=[
            pltpu.VMEM((n_chunks, _C), jnp.int32),
            pltpu.VMEM((n_per_w,), jnp.int32),
            pltpu.VMEM((16,), jnp.float32),
            pltpu.VMEM((16,), jnp.float32),
            [pltpu.VMEM((_C, width), jnp.float32) for _ in range(_NBUF)],
            [pltpu.SemaphoreType.DMA for _ in range(_NBUF)],
            [pltpu.SemaphoreType.DMA for _ in range(_NBUF)],
        ],
        compiler_params=pltpu.CompilerParams(needs_layout_passes=False),
    )
    def sc_gather(table_hbm, idx_hbm, tgt_hbm,
                  out_hbm, lpart_hbm, tpart_hbm,
                  idx_v, tgt_v, pl_v, pv_v, bufs, gsems, ssems):
        cid = lax.axis_index("c")
        sid = lax.axis_index("s")
        wid = sid * _NC + cid
        base = wid * n_per_w

        pltpu.sync_copy(idx_hbm.at[wid], idx_v)
        pltpu.sync_copy(tgt_hbm.at[wid], tgt_v)
        pl_v[...] = jnp.zeros((16,), jnp.float32)
        pv_v[...] = jnp.zeros((16,), jnp.float32)

        lane = lax.iota(jnp.int32, 16)
        msk_c = lane < _C
        neg_big = jnp.full((16,), jnp.finfo(jnp.float32).min, jnp.float32)
        zeros = jnp.zeros((16,), jnp.float32)

        def start_gather(k, b):
            pltpu.async_copy(table_hbm.at[idx_v.at[k]], bufs[b], gsems[b])

        def wait_gather(b):
            pltpu.make_async_copy(
                out_hbm.at[pl.ds(base, _C)], bufs[b], gsems[b]).wait()

        def start_scatter(k, b):
            pltpu.async_copy(
                bufs[b], out_hbm.at[pl.ds(base + k * _C, _C)], ssems[b])

        def wait_scatter(b):
            pltpu.make_async_copy(
                bufs[b], out_hbm.at[pl.ds(base, _C)], ssems[b]).wait()

        def compute(k, b):
            buf = bufs[b]
            tok = jnp.minimum(k * _C + lane, n_per_w - 1)
            tgts = plsc.load_gather(tgt_v, [tok])
            rowl = jnp.minimum(lane, _C - 1)
            tv = plsc.load_gather(buf, [rowl, tgts])
            pv_v[...] += jnp.where(msk_c, tv, jnp.float32(0.0))

            def p1(j, ms):
                return tuple(
                    jnp.maximum(ms[r], buf[r, pl.ds(j * 16, 16)])
                    for r in range(_C))

            m = lax.fori_loop(0, nvec, p1, (neg_big,) * _C, unroll=8)

            def p2(j, ss):
                return tuple(
                    ss[r] + jnp.exp(buf[r, pl.ds(j * 16, 16)] - m[r])
                    for r in range(_C))

            s = lax.fori_loop(0, nvec, p2, (zeros,) * _C, unroll=8)

            mvec = zeros
            svec = jnp.full((16,), jnp.float32(1.0), jnp.float32)
            for r in range(_C):
                mr = jnp.max(m[r])
                sr = jnp.sum(s[r] * jnp.exp(m[r] - mr))
                mvec = jnp.where(lane == r, mr, mvec)
                svec = jnp.where(lane == r, sr, svec)
            lse = mvec + _vlog(svec)
            pl_v[...] += jnp.where(msk_c, lse, jnp.float32(0.0))

        for b in range(_NBUF):
            start_gather(b, b)

        def loop_body(go, carry):
            for b in range(_NBUF):
                k = go * _NBUF + b
                wait_gather(b)
                start_scatter(k, b)
                compute(k, b)
                wait_scatter(b)
                start_gather(k + _NBUF, b)
            return carry

        n_full = n_chunks // _NBUF - 1
        lax.fori_loop(0, n_full, loop_body, 0, unroll=False)

        for k in range(n_full * _NBUF, n_chunks):
            b = k % _NBUF
            wait_gather(b)
            start_scatter(k, b)
            compute(k, b)
            wait_scatter(b)
            if k + _NBUF < n_chunks:
                start_gather(k + _NBUF, b)

        pltpu.sync_copy(pl_v, lpart_hbm.at[wid])
        pltpu.sync_copy(pv_v, tpart_hbm.at[wid])

    return sc_gather


def kernel(idx, targets, table):
    vocab, width = table.shape
    n_tok = idx.shape[0] * idx.shape[1]
    n_per_w = n_tok // _NW
    idx3 = idx.reshape(_NW, n_per_w // _C, _C).astype(jnp.int32)
    tgt2 = targets.reshape(_NW, n_per_w).astype(jnp.int32)

    logits2, lse_parts, tval_parts = _make_sc_gather(n_tok, vocab, width)(
        table, idx3, tgt2)
    loss = (jnp.sum(lse_parts) - jnp.sum(tval_parts)) / jnp.float32(n_tok)
    return (logits2, loss)
